# baseline (speedup 1.0000x reference)
import jax
import jax.numpy as jnp
from jax import lax
from jax.experimental import pallas as pl
from jax.experimental.pallas import tpu as pltpu
from jax.experimental.pallas import tpu_sc as plsc

N = 50000
F = 128
P = 25000
K = 16

NC = 2
NS = 16
NW = NC * NS

CPTS = 8
GCHUNK = P // CPTS
NCHUNK = 100
MAXSTART = GCHUNK - NCHUNK
ROWS = CPTS * K
LANES = 16
NBUF = 4
NOBUF = 2


def _pool_body(features_hbm, idx_hbm, out_hbm, rows_v, out_v,
               ib0, ib1, ib2, ib3, is0, is1, is2, is3,
               gs0, gs1, gs2, gs3, os0, os1):
    wid = lax.axis_index("s") * NC + lax.axis_index("c")
    start = jnp.minimum((wid * 6250) >> 6, MAXSTART)
    idxbufs = (ib0, ib1, ib2, ib3)
    isems = (is0, is1, is2, is3)
    gsems = (gs0, gs1, gs2, gs3)
    osems = (os0, os1)

    def idx_fetch_start(g, b):
        pltpu.make_async_copy(
            idx_hbm.at[start + g], idxbufs[b], isems[b]
        ).start()

    def idx_wait(b):
        pltpu.make_async_copy(
            idx_hbm.at[start], idxbufs[b], isems[b]
        ).wait()

    def gather_start(b):
        pltpu.make_async_copy(
            features_hbm.at[idxbufs[b].at[0]], rows_v.at[b], gsems[b]
        ).start()

    def gather_wait(b):
        pltpu.make_async_copy(
            features_hbm.at[idxbufs[b].at[0]], rows_v.at[b], gsems[b]
        ).wait()

    def store_start(g, ob):
        pltpu.make_async_copy(
            out_v.at[ob],
            out_hbm.at[pl.ds((start + g) * CPTS, CPTS)],
            osems[ob],
        ).start()

    def store_wait(ob):
        pltpu.make_async_copy(
            out_v.at[ob], out_hbm.at[pl.ds(0, CPTS)], osems[ob]
        ).wait()

    for b in range(NBUF):
        idx_fetch_start(b, b)
    for b in range(NBUF):
        idx_wait(b)
        gather_start(b)

    @pl.loop(0, NCHUNK, step=NBUF)
    def _chunks(g4):
        for b in range(NBUF):
            gg = g4 + b
            ob = b % NOBUF
            gather_wait(b)
            nxt = gg + NBUF

            @pl.when(nxt < NCHUNK)
            def _():
                idx_fetch_start(nxt, b)

            @pl.when(gg >= NOBUF)
            def _():
                store_wait(ob)

            @plsc.parallel_loop(0, CPTS, unroll=4)
            def _pts(i):
                r0 = i * K
                for j in range(F // LANES):
                    col = pl.ds(j * LANES, LANES)
                    acc = None
                    for k0 in range(0, K, 4):
                        v0 = rows_v[b, r0 + k0, col]
                        v1 = rows_v[b, r0 + k0 + 1, col]
                        v2 = rows_v[b, r0 + k0 + 2, col]
                        v3 = rows_v[b, r0 + k0 + 3, col]
                        m = jnp.maximum(jnp.maximum(v0, v1),
                                        jnp.maximum(v2, v3))
                        acc = m if acc is None else jnp.maximum(acc, m)
                    out_v[ob, i, col] = acc

            store_start(gg, ob)

            @pl.when(nxt < NCHUNK)
            def _():
                idx_wait(b)
                gather_start(b)

    store_wait(0)
    store_wait(1)


_pool_kernel = pl.kernel(
    _pool_body,
    mesh=plsc.VectorSubcoreMesh(core_axis_name="c", subcore_axis_name="s"),
    out_type=jax.ShapeDtypeStruct((P, F), jnp.float32),
    scratch_types=[
        pltpu.VMEM((NBUF, ROWS, F), jnp.float32),
        pltpu.VMEM((NOBUF, CPTS, F), jnp.float32),
        pltpu.VMEM((1, 128), jnp.int32),
        pltpu.VMEM((1, 128), jnp.int32),
        pltpu.VMEM((1, 128), jnp.int32),
        pltpu.VMEM((1, 128), jnp.int32),
        pltpu.SemaphoreType.DMA,
        pltpu.SemaphoreType.DMA,
        pltpu.SemaphoreType.DMA,
        pltpu.SemaphoreType.DMA,
        pltpu.SemaphoreType.DMA,
        pltpu.SemaphoreType.DMA,
        pltpu.SemaphoreType.DMA,
        pltpu.SemaphoreType.DMA,
        pltpu.SemaphoreType.DMA,
        pltpu.SemaphoreType.DMA,
    ],
)


def kernel(points, features, neighbor_indices):
    del points
    idx = neighbor_indices.astype(jnp.int32)
    idx2 = idx.reshape(GCHUNK, 1, 128)
    return _pool_kernel(features, idx2)


# speedup vs baseline: 1.3223x; 1.3223x over previous
import jax
import jax.numpy as jnp
from jax import lax
from jax.experimental import pallas as pl
from jax.experimental.pallas import tpu as pltpu
from jax.experimental.pallas import tpu_sc as plsc

N = 50000
F = 128
P = 25000
K = 16

NC = 2
NS = 16
NW = NC * NS

CPTS = 8
GCHUNK = P // CPTS
NCHUNK = 100
MAXSTART = GCHUNK - NCHUNK
ROWS = CPTS * K
LANES = 16
NBUF = 4
NOBUF = 2


def _pool_body(features_hbm, idx_hbm, out_hbm, rows_v, out_v,
               ib0, ib1, ib2, ib3, is0, is1, is2, is3,
               gs0, gs1, gs2, gs3, os0, os1):
    wid = lax.axis_index("s") * NC + lax.axis_index("c")
    start = jnp.minimum((wid * 6250) >> 6, MAXSTART)
    idxbufs = (ib0, ib1, ib2, ib3)
    isems = (is0, is1, is2, is3)
    gsems = (gs0, gs1, gs2, gs3)
    osems = (os0, os1)

    def idx_fetch_start(g, b):
        pltpu.make_async_copy(
            idx_hbm.at[start + g], idxbufs[b], isems[b]
        ).start()

    def idx_wait(b):
        pltpu.make_async_copy(
            idx_hbm.at[start], idxbufs[b], isems[b]
        ).wait()

    def gather_start(b):
        pltpu.make_async_copy(
            features_hbm.at[idxbufs[b].at[0]], rows_v.at[b], gsems[b]
        ).start()

    def gather_wait(b):
        pltpu.make_async_copy(
            features_hbm.at[idxbufs[b].at[0]], rows_v.at[b], gsems[b]
        ).wait()

    def store_start(g, ob):
        pltpu.make_async_copy(
            out_v.at[ob],
            out_hbm.at[pl.ds((start + g) * CPTS, CPTS)],
            osems[ob],
        ).start()

    def store_wait(ob):
        pltpu.make_async_copy(
            out_v.at[ob], out_hbm.at[pl.ds(0, CPTS)], osems[ob]
        ).wait()

    for b in range(NBUF):
        idx_fetch_start(b, b)
    for b in range(NBUF):
        idx_wait(b)
        gather_start(b)

    @pl.loop(0, NCHUNK, step=NBUF)
    def _chunks(g4):
        for b in range(NBUF):
            gg = g4 + b
            ob = b % NOBUF
            gather_wait(b)
            nxt = gg + NBUF

            @pl.when(nxt < NCHUNK)
            def _():
                idx_fetch_start(nxt, b)

            @pl.when(gg >= NOBUF)
            def _():
                store_wait(ob)

            @plsc.parallel_loop(0, CPTS * (F // LANES), unroll=2)
            def _pts(t):
                i = t // (F // LANES)
                j = t % (F // LANES)
                r0 = i * K
                col = pl.ds(j * LANES, LANES)
                acc = None
                for k0 in range(0, K, 4):
                    v0 = rows_v[b, r0 + k0, col]
                    v1 = rows_v[b, r0 + k0 + 1, col]
                    v2 = rows_v[b, r0 + k0 + 2, col]
                    v3 = rows_v[b, r0 + k0 + 3, col]
                    m = jnp.maximum(jnp.maximum(v0, v1),
                                    jnp.maximum(v2, v3))
                    acc = m if acc is None else jnp.maximum(acc, m)
                out_v[ob, i, col] = acc

            store_start(gg, ob)

            @pl.when(nxt < NCHUNK)
            def _():
                idx_wait(b)
                gather_start(b)

    store_wait(0)
    store_wait(1)


_pool_kernel = pl.kernel(
    _pool_body,
    mesh=plsc.VectorSubcoreMesh(core_axis_name="c", subcore_axis_name="s"),
    out_type=jax.ShapeDtypeStruct((P, F), jnp.float32),
    scratch_types=[
        pltpu.VMEM((NBUF, ROWS, F), jnp.float32),
        pltpu.VMEM((NOBUF, CPTS, F), jnp.float32),
        pltpu.VMEM((1, 128), jnp.int32),
        pltpu.VMEM((1, 128), jnp.int32),
        pltpu.VMEM((1, 128), jnp.int32),
        pltpu.VMEM((1, 128), jnp.int32),
        pltpu.SemaphoreType.DMA,
        pltpu.SemaphoreType.DMA,
        pltpu.SemaphoreType.DMA,
        pltpu.SemaphoreType.DMA,
        pltpu.SemaphoreType.DMA,
        pltpu.SemaphoreType.DMA,
        pltpu.SemaphoreType.DMA,
        pltpu.SemaphoreType.DMA,
        pltpu.SemaphoreType.DMA,
        pltpu.SemaphoreType.DMA,
    ],
)


def kernel(points, features, neighbor_indices):
    del points
    idx = neighbor_indices.astype(jnp.int32)
    idx2 = idx.reshape(GCHUNK, 1, 128)
    return _pool_kernel(features, idx2)
